# baseline (speedup 1.0000x reference)
import functools

import jax
import jax.numpy as jnp
from jax import lax
from jax.experimental import pallas as pl
from jax.experimental.pallas import tpu as pltpu
from jax.experimental.pallas import tpu_sc as plsc

V = 1000000
D = 128
EMB = 2048
B = 4096

_NC = 2
_NS = 16
_NW = _NC * _NS
_BPW = B // _NW


def _make_gather(bc):
    bpw = bc // _NW
    mesh = plsc.VectorSubcoreMesh(core_axis_name="c", subcore_axis_name="s")

    @functools.partial(
        pl.kernel,
        mesh=mesh,
        out_type=(
            jax.ShapeDtypeStruct((bc, D), jnp.float32),
            jax.ShapeDtypeStruct((bc, D), jnp.float32),
        ),
        scratch_types=[
            pltpu.VMEM((bpw,), jnp.int32),
            pltpu.VMEM((bpw, D), jnp.float32),
            pltpu.VMEM((bpw, D), jnp.float32),
            pltpu.SemaphoreType.DMA,
            pltpu.SemaphoreType.DMA,
        ],
    )
    def gather_k(idx_hbm, ent_hbm, rel_hbm, ent_out, rel_out,
                 idx_v, ent_v, rel_v, sem_e, sem_r):
        wid = lax.axis_index("s") * _NC + lax.axis_index("c")
        base = wid * bpw
        pltpu.sync_copy(idx_hbm.at[pl.ds(base, bpw)], idx_v)
        ce = pltpu.async_copy(ent_hbm.at[idx_v], ent_v, sem_e)
        cr = pltpu.async_copy(rel_hbm.at[idx_v], rel_v, sem_r)
        ce.wait()
        cr.wait()
        pltpu.sync_copy(ent_v, ent_out.at[pl.ds(base, bpw)])
        pltpu.sync_copy(rel_v, rel_out.at[pl.ds(base, bpw)])

    return gather_k


_gather = _make_gather(B)

_BS = 512


def _mm_body(ent_ref, rel_ref, w_ref, b_ref, out_ref):
    w = w_ref[:]
    bias = b_ref[:]
    out_ref[:, 0, :] = (
        jnp.dot(ent_ref[:], w, preferred_element_type=jnp.float32) + bias
    )
    out_ref[:, 1, :] = (
        jnp.dot(rel_ref[:], w, preferred_element_type=jnp.float32) + bias
    )


def _mm_chain_body(prev_ref, ent_ref, rel_ref, w_ref, b_ref, out_ref):
    del prev_ref
    _mm_body(ent_ref, rel_ref, w_ref, b_ref, out_ref)


_NCHUNK = 2
_BC = B // _NCHUNK
_NBLK = _BC // _BS
_gather_c = _make_gather(_BC)


def _make_mm(chunk):
    data_specs = [
        pl.BlockSpec((_BS, D), lambda i: (i, 0)),
        pl.BlockSpec((_BS, D), lambda i: (i, 0)),
        pl.BlockSpec((D, EMB), lambda i: (0, 0)),
        pl.BlockSpec((1, EMB), lambda i: (0, 0)),
    ]
    out_spec = pl.BlockSpec(
        (_BS, 2, EMB), lambda i, c=chunk: (c * _NBLK + i, 0, 0)
    )
    out_shape = jax.ShapeDtypeStruct((B, 2, EMB), jnp.float32)
    if chunk == 0:
        return pl.pallas_call(
            _mm_body,
            grid=(_NBLK,),
            in_specs=data_specs,
            out_specs=out_spec,
            out_shape=out_shape,
        )
    return pl.pallas_call(
        _mm_chain_body,
        grid=(_NBLK,),
        in_specs=[pl.BlockSpec(memory_space=pl.ANY)] + data_specs,
        out_specs=out_spec,
        out_shape=out_shape,
        input_output_aliases={0: 0},
    )


_mms = [_make_mm(c) for c in range(_NCHUNK)]


def kernel(question_id, ent_table, rel_table, W, b):
    b2 = b.reshape(1, EMB)
    gathered = []
    for c in range(_NCHUNK):
        idx_c = lax.slice(question_id, (c * _BC,), ((c + 1) * _BC,))
        gathered.append(_gather_c(idx_c, ent_table, rel_table))
    out = None
    for c in range(_NCHUNK):
        ent_c, rel_c = gathered[c]
        if c == 0:
            out = _mms[0](ent_c, rel_c, W, b2)
        else:
            out = _mms[c](out, ent_c, rel_c, W, b2)
    return out


# speedup vs baseline: 1.0656x; 1.0656x over previous
import functools

import jax
import jax.numpy as jnp
from jax import lax
from jax.experimental import pallas as pl
from jax.experimental.pallas import tpu as pltpu
from jax.experimental.pallas import tpu_sc as plsc

V = 1000000
D = 128
EMB = 2048
B = 4096

_NC = 2
_NS = 16
_NW = _NC * _NS
_BPW = B // _NW


def _make_gather():
    mesh = plsc.VectorSubcoreMesh(core_axis_name="c", subcore_axis_name="s")

    @functools.partial(
        pl.kernel,
        mesh=mesh,
        out_type=(
            jax.ShapeDtypeStruct((B, D), jnp.float32),
            jax.ShapeDtypeStruct((B, D), jnp.float32),
        ),
        scratch_types=[
            pltpu.VMEM((_BPW,), jnp.int32),
            pltpu.VMEM((_BPW, D), jnp.float32),
            pltpu.VMEM((_BPW, D), jnp.float32),
            pltpu.SemaphoreType.DMA,
            pltpu.SemaphoreType.DMA,
            pltpu.SemaphoreType.DMA,
            pltpu.SemaphoreType.DMA,
        ],
    )
    def gather_k(idx_hbm, ent_hbm, rel_hbm, ent_out, rel_out,
                 idx_v, ent_v, rel_v, sem_e, sem_r, sem_se, sem_sr):
        wid = lax.axis_index("s") * _NC + lax.axis_index("c")
        base = wid * _BPW
        pltpu.sync_copy(idx_hbm.at[pl.ds(base, _BPW)], idx_v)
        ce = pltpu.async_copy(ent_hbm.at[idx_v], ent_v, sem_e)
        cr = pltpu.async_copy(rel_hbm.at[idx_v], rel_v, sem_r)
        ce.wait()
        se = pltpu.async_copy(ent_v, ent_out.at[pl.ds(base, _BPW)], sem_se)
        cr.wait()
        sr = pltpu.async_copy(rel_v, rel_out.at[pl.ds(base, _BPW)], sem_sr)
        se.wait()
        sr.wait()

    return gather_k


_gather = _make_gather()

_BS = 512


def _mm_body(ent_ref, rel_ref, w_ref, b_ref, out_ref):
    w = w_ref[:]
    bias = b_ref[:]
    out_ref[:, 0, :] = (
        jnp.dot(ent_ref[:], w, preferred_element_type=jnp.float32) + bias
    )
    out_ref[:, 1, :] = (
        jnp.dot(rel_ref[:], w, preferred_element_type=jnp.float32) + bias
    )


_mm = pl.pallas_call(
    _mm_body,
    grid=(B // _BS,),
    in_specs=[
        pl.BlockSpec((_BS, D), lambda i: (i, 0)),
        pl.BlockSpec((_BS, D), lambda i: (i, 0)),
        pl.BlockSpec((D, EMB), lambda i: (0, 0)),
        pl.BlockSpec((1, EMB), lambda i: (0, 0)),
    ],
    out_specs=pl.BlockSpec((_BS, 2, EMB), lambda i: (i, 0, 0)),
    out_shape=jax.ShapeDtypeStruct((B, 2, EMB), jnp.float32),
)


def kernel(question_id, ent_table, rel_table, W, b):
    ent_g, rel_g = _gather(question_id, ent_table, rel_table)
    return _mm(ent_g, rel_g, W, b.reshape(1, EMB))
